# baseline (speedup 1.0000x reference)
import functools

import jax
import jax.numpy as jnp
from jax import lax
from jax.experimental import pallas as pl
from jax.experimental.pallas import tpu as pltpu
from jax.experimental.pallas import tpu_sc as plsc

N = 10000
E = 320000
D = 128
G = 64

NC = 2
NS = 16
NW = NC * NS
NPAD = 10240
RPT = NPAD // NS
CHUNK = 128
EPW = 10240
EPAD = EPW * NW
NCHUNK = EPW // CHUNK
BM = 512
POOL_RB = 32

_mesh = plsc.VectorSubcoreMesh(core_axis_name="c", subcore_axis_name="s")


@functools.partial(
    pl.kernel,
    out_type=jax.ShapeDtypeStruct((NC * NPAD,), jnp.float32),
    mesh=_mesh,
    compiler_params=pltpu.CompilerParams(needs_layout_passes=False),
    scratch_types=[
        pltpu.VMEM_SHARED((NPAD,), jnp.float32),
        pltpu.VMEM((CHUNK,), jnp.float32),
        pltpu.VMEM((CHUNK,), jnp.int32),
    ],
)
def _sc_degree(dst_hbm, zero1_hbm, out_hbm, acc, onesv, dstv):
    c = lax.axis_index("c")
    s = lax.axis_index("s")
    w = s * NC + c
    for j in range(CHUNK // 16):
        onesv[pl.ds(j * 16, 16)] = jnp.ones((16,), jnp.float32)
    pltpu.sync_copy(zero1_hbm, acc.at[pl.ds(s * RPT, RPT)])
    plsc.subcore_barrier()

    def body(g, carry):
        eb = w * EPW + g * CHUNK
        pltpu.sync_copy(dst_hbm.at[pl.ds(eb, CHUNK)], dstv)
        pltpu.sync_copy(onesv, acc.at[dstv], add=True)
        return carry

    lax.fori_loop(0, NCHUNK, body, 0)
    plsc.subcore_barrier()
    pltpu.sync_copy(acc.at[pl.ds(s * RPT, RPT)],
                    out_hbm.at[pl.ds(c * NPAD + s * RPT, RPT)])


EPW1 = EPAD // NS
NCHUNK1 = EPW1 // CHUNK


def _make_prop(npass):
    scratch = [
        pltpu.VMEM_SHARED((NPAD, D), jnp.float32),
        pltpu.VMEM((NCHUNK1 // 2, CHUNK), jnp.int32),
        pltpu.VMEM((NCHUNK1 // 4, CHUNK), jnp.int32),
        pltpu.VMEM((2, CHUNK, D), jnp.float32),
        pltpu.SemaphoreType.DMA,
        pltpu.SemaphoreType.DMA,
    ]

    @functools.partial(
        pl.kernel,
        out_type=jax.ShapeDtypeStruct((npass * NPAD, D), jnp.float32),
        mesh=_mesh,
        compiler_params=pltpu.CompilerParams(needs_layout_passes=False),
        scratch_types=scratch,
    )
    def prop(*refs):
        ys = refs[:npass]
        src3, dst3, zero_hbm, out_hbm = refs[npass:npass + 4]
        acc, src2, dst2, rows, sem0, sem1 = refs[npass + 4:]
        c = lax.axis_index("c")
        s = lax.axis_index("s")

        @pl.when(c == 0)
        def _():
            for k in range(npass):
                y = ys[k]
                pltpu.sync_copy(zero_hbm, acc.at[pl.ds(s * RPT, RPT)])
                plsc.subcore_barrier()

                def start_g(gl, b, sem):
                    pltpu.async_copy(y.at[src2.at[gl]], rows.at[b], sem)

                def wait_g(b, sem):
                    pltpu.make_async_copy(y.at[src2.at[0]], rows.at[b],
                                          sem).wait()

                def scat(j, b):
                    pltpu.sync_copy(rows.at[b], acc.at[dst2.at[j]],
                                    add=True)

                def step(gl0, j0):
                    wait_g(0, sem0)
                    scat(j0, 0)
                    start_g(gl0 + 2, 0, sem0)
                    wait_g(1, sem1)
                    scat(j0 + 1, 1)
                    start_g(gl0 + 3, 1, sem1)

                for q in range(4):
                    if q % 2 == 0:
                        pltpu.sync_copy(
                            src3.at[s, pl.ds((q // 2) * 80, 80)], src2)
                    pltpu.sync_copy(dst3.at[s, pl.ds(q * 40, 40)], dst2)
                    loc = (q % 2) * 40
                    start_g(loc, 0, sem0)
                    start_g(loc + 1, 1, sem1)

                    def body(p, carry, loc=loc):
                        step(loc + 2 * p, 2 * p)
                        return carry

                    lax.fori_loop(0, 19, body, 0)
                    wait_g(0, sem0)
                    scat(38, 0)
                    wait_g(1, sem1)
                    scat(39, 1)
                plsc.subcore_barrier()
                pltpu.sync_copy(
                    acc.at[pl.ds(s * RPT, RPT)],
                    out_hbm.at[pl.ds(k * NPAD + s * RPT, RPT)])

    return prop


_prop1 = _make_prop(1)
_prop2 = _make_prop(2)
_prop4 = _make_prop(4)


RPW = NPAD // NW


@functools.partial(
    pl.kernel,
    out_type=jax.ShapeDtypeStruct((NW, G + 1, 4 * D), jnp.float32),
    mesh=_mesh,
    compiler_params=pltpu.CompilerParams(needs_layout_passes=False),
    scratch_types=[
        pltpu.VMEM((RPW,), jnp.int32),
        pltpu.VMEM((POOL_RB, 4 * D), jnp.float32),
        pltpu.VMEM((G + 1, 4 * D), jnp.float32),
    ],
)
def _sc_pool(h3_hbm, batch_hbm, out_hbm, bchunk, rbuf, acc):
    c = lax.axis_index("c")
    s = lax.axis_index("s")
    w = s * NC + c
    pltpu.sync_copy(batch_hbm.at[pl.ds(w * RPW, RPW)], bchunk)
    iota = lax.iota(jnp.int32, 16)
    zeros16 = jnp.zeros((16,), jnp.float32)
    for g in range(G + 1):
        for j in range(4 * D // 16):
            acc[g, pl.ds(j * 16, 16)] = zeros16

    def chunk_body(ci, carry):
        r0 = ci * POOL_RB
        pltpu.sync_copy(h3_hbm.at[pl.ds(w * RPW + r0, POOL_RB)], rbuf)

        def row_body(i, carry2):
            li = r0 + i
            seg = plsc.load_gather(bchunk, [jnp.full((16,), li, jnp.int32)])
            for j in range(4 * D // 16):
                col = j * 16 + iota
                rv = rbuf[i, pl.ds(j * 16, 16)]
                cur = plsc.load_gather(acc, [seg, col])
                plsc.store_scatter(acc, [seg, col], jnp.maximum(cur, rv))
            return carry2

        lax.fori_loop(0, POOL_RB, row_body, 0)
        return carry

    lax.fori_loop(0, RPW // POOL_RB, chunk_body, 0)
    pltpu.sync_copy(acc, out_hbm.at[w])


def _tc_first(x_pad, cnt_t, W1):
    def body(x_ref, cnt_ref, w_ref, y1_ref, dis_ref):
        i = pl.program_id(0)
        rows = lax.broadcasted_iota(jnp.int32, (BM, 1), 0) + i * BM
        deg = cnt_ref[:, 0:1] + cnt_ref[:, 1:2] + 1.0
        dis = lax.rsqrt(deg) * jnp.where(rows < N, 1.0, 0.0)
        dis_ref[...] = dis
        h = jnp.dot(x_ref[...], w_ref[...],
                    preferred_element_type=jnp.float32)
        y1_ref[...] = h * dis

    return pl.pallas_call(
        body,
        grid=(NPAD // BM,),
        in_specs=[
            pl.BlockSpec((BM, D), lambda i: (i, 0)),
            pl.BlockSpec((BM, 2), lambda i: (i, 0)),
            pl.BlockSpec((D, D), lambda i: (0, 0)),
        ],
        out_specs=[
            pl.BlockSpec((BM, D), lambda i: (i, 0)),
            pl.BlockSpec((BM, 1), lambda i: (i, 0)),
        ],
        out_shape=[
            jax.ShapeDtypeStruct((NPAD, D), jnp.float32),
            jax.ShapeDtypeStruct((NPAD, 1), jnp.float32),
        ],
    )(x_pad, cnt_t, W1)


def _tc_mid1(S, y1, dis, b1, W2):
    def body(s_ref, y_ref, dis_ref, b_ref, w_ref, out_ref):
        d = dis_ref[...]
        h1 = jnp.maximum(
            (s_ref[...] + y_ref[...]) * d + b_ref[...], 0.0)
        y2 = jnp.dot(h1, w_ref[...], preferred_element_type=jnp.float32) * d
        out_ref[0] = y2[:, 0:D]
        out_ref[1] = y2[:, D:2 * D]

    return pl.pallas_call(
        body,
        grid=(NPAD // BM,),
        in_specs=[
            pl.BlockSpec((BM, D), lambda i: (i, 0)),
            pl.BlockSpec((BM, D), lambda i: (i, 0)),
            pl.BlockSpec((BM, 1), lambda i: (i, 0)),
            pl.BlockSpec((1, D), lambda i: (0, 0)),
            pl.BlockSpec((D, 2 * D), lambda i: (0, 0)),
        ],
        out_specs=pl.BlockSpec((2, BM, D), lambda i: (0, i, 0)),
        out_shape=jax.ShapeDtypeStruct((2, NPAD, D), jnp.float32),
    )(S, y1, dis, b1, W2)


def _tc_mid2(S2, y2, dis, b2, W3):
    def body(s_ref, y_ref, dis_ref, b_ref, w_ref, out_ref):
        d = dis_ref[...]
        z = jnp.concatenate(
            [(s_ref[0] + y_ref[0]) * d,
             (s_ref[1] + y_ref[1]) * d], axis=1)
        h2 = jnp.maximum(z + b_ref[...], 0.0)
        y3 = jnp.dot(h2, w_ref[...], preferred_element_type=jnp.float32) * d
        for k in range(4):
            out_ref[k] = y3[:, k * D:(k + 1) * D]

    return pl.pallas_call(
        body,
        grid=(NPAD // BM,),
        in_specs=[
            pl.BlockSpec((2, BM, D), lambda i: (0, i, 0)),
            pl.BlockSpec((2, BM, D), lambda i: (0, i, 0)),
            pl.BlockSpec((BM, 1), lambda i: (i, 0)),
            pl.BlockSpec((1, 2 * D), lambda i: (0, 0)),
            pl.BlockSpec((2 * D, 4 * D), lambda i: (0, 0)),
        ],
        out_specs=pl.BlockSpec((4, BM, D), lambda i: (0, i, 0)),
        out_shape=jax.ShapeDtypeStruct((4, NPAD, D), jnp.float32),
    )(S2, y2, dis, b2, W3)


def _tc_last(S3, y3, dis, b3):
    def body(s_ref, y_ref, dis_ref, b_ref, out_ref):
        d = dis_ref[...]
        z = jnp.concatenate(
            [(s_ref[k] + y_ref[k]) * d for k in range(4)],
            axis=1)
        out_ref[...] = jnp.maximum(z + b_ref[...], 0.0)

    return pl.pallas_call(
        body,
        grid=(NPAD // BM,),
        in_specs=[
            pl.BlockSpec((4, BM, D), lambda i: (0, i, 0)),
            pl.BlockSpec((4, BM, D), lambda i: (0, i, 0)),
            pl.BlockSpec((BM, 1), lambda i: (i, 0)),
            pl.BlockSpec((1, 4 * D), lambda i: (0, 0)),
        ],
        out_specs=pl.BlockSpec((BM, 4 * D), lambda i: (i, 0)),
        out_shape=jax.ShapeDtypeStruct((NPAD, 4 * D), jnp.float32),
    )(S3, y3, dis, b3)


def _tc_head(parts, Wg1, bg1, Wg2, bg2):
    def body(p_ref, w1_ref, b1_ref, w2_ref, b2_ref, out_ref):
        pooled = jnp.max(p_ref[:, 0:G, :], axis=0)
        g = jnp.maximum(
            jnp.dot(pooled, w1_ref[...],
                    preferred_element_type=jnp.float32) + b1_ref[...], 0.0)
        out_ref[...] = (
            jnp.dot(g, w2_ref[...], preferred_element_type=jnp.float32)
            + b2_ref[...])

    return pl.pallas_call(
        body,
        out_shape=jax.ShapeDtypeStruct((G, 1), jnp.float32),
    )(parts, Wg1, bg1, Wg2, bg2)


def kernel(x, edge_index, batch, W1, b1, W2, b2, W3, b3, Wg1, bg1, Wg2, bg2):
    src = edge_index[0].astype(jnp.int32)
    dst = edge_index[1].astype(jnp.int32)
    epad = jnp.full((EPAD - E,), N, dtype=jnp.int32)
    src_p = jnp.concatenate([src, epad])
    dst_p = jnp.concatenate([dst, epad])
    x_pad = jnp.pad(x, ((0, NPAD - N), (0, 0)))
    zeros1 = jnp.zeros((RPT,), jnp.float32)
    zeros2 = jnp.zeros((RPT, D), jnp.float32)

    batch_p = jnp.concatenate(
        [batch.astype(jnp.int32),
         jnp.full((NPAD - N,), G, dtype=jnp.int32)])

    cnt = _sc_degree(dst_p, zeros1)
    cnt_t = cnt.reshape(NC, NPAD).T

    src3 = src_p.reshape(NS, NCHUNK1, CHUNK)
    dst3 = dst_p.reshape(NS, NCHUNK1, CHUNK)

    y1, dis = _tc_first(x_pad, cnt_t, W1)
    S1 = _prop1(y1, src3, dst3, zeros2)
    y2 = _tc_mid1(S1, y1, dis, b1.reshape(1, D), W2)

    S2 = _prop2(y2[0], y2[1], src3, dst3, zeros2).reshape(2, NPAD, D)
    y3 = _tc_mid2(S2, y2, dis, b2.reshape(1, 2 * D), W3)

    S3 = _prop4(y3[0], y3[1], y3[2], y3[3], src3, dst3,
                zeros2).reshape(4, NPAD, D)
    h3 = _tc_last(S3, y3, dis, b3.reshape(1, 4 * D))

    parts = _sc_pool(h3, batch_p)
    return _tc_head(parts, Wg1, bg1.reshape(1, 1024), Wg2,
                    bg2.reshape(1, 1))


# speedup vs baseline: 1.6549x; 1.6549x over previous
import functools

import jax
import jax.numpy as jnp
from jax import lax
from jax.experimental import pallas as pl
from jax.experimental.pallas import tpu as pltpu
from jax.experimental.pallas import tpu_sc as plsc

N = 10000
E = 320000
D = 128
G = 64

NC = 2
NS = 16
NW = NC * NS
NPAD = 10240
RPT = NPAD // NS
CHUNK = 128
EPW = 10240
EPAD = EPW * NW
NCHUNK = EPW // CHUNK
BM = 512
POOL_RB = 32

_mesh = plsc.VectorSubcoreMesh(core_axis_name="c", subcore_axis_name="s")


@functools.partial(
    pl.kernel,
    out_type=jax.ShapeDtypeStruct((NC * NPAD,), jnp.float32),
    mesh=_mesh,
    compiler_params=pltpu.CompilerParams(needs_layout_passes=False),
    scratch_types=[
        pltpu.VMEM_SHARED((NPAD,), jnp.float32),
        pltpu.VMEM((CHUNK,), jnp.float32),
        pltpu.VMEM((CHUNK,), jnp.int32),
    ],
)
def _sc_degree(dst_hbm, zero1_hbm, out_hbm, acc, onesv, dstv):
    c = lax.axis_index("c")
    s = lax.axis_index("s")
    w = s * NC + c
    for j in range(CHUNK // 16):
        onesv[pl.ds(j * 16, 16)] = jnp.ones((16,), jnp.float32)
    pltpu.sync_copy(zero1_hbm, acc.at[pl.ds(s * RPT, RPT)])
    plsc.subcore_barrier()

    def body(g, carry):
        eb = w * EPW + g * CHUNK
        pltpu.sync_copy(dst_hbm.at[pl.ds(eb, CHUNK)], dstv)
        pltpu.sync_copy(onesv, acc.at[dstv], add=True)
        return carry

    lax.fori_loop(0, NCHUNK, body, 0)
    plsc.subcore_barrier()
    pltpu.sync_copy(acc.at[pl.ds(s * RPT, RPT)],
                    out_hbm.at[pl.ds(c * NPAD + s * RPT, RPT)])


def _make_prop_es(npass):
    scratch = [
        pltpu.VMEM_SHARED((NPAD, D), jnp.float32),
        pltpu.VMEM((NCHUNK, CHUNK), jnp.int32),
        pltpu.VMEM((NCHUNK // 2, CHUNK), jnp.int32),
        pltpu.VMEM((2, CHUNK, D), jnp.float32),
        pltpu.SemaphoreType.DMA,
        pltpu.SemaphoreType.DMA,
        pltpu.SemaphoreType.DMA,
        pltpu.SemaphoreType.DMA,
    ]

    @functools.partial(
        pl.kernel,
        out_type=jax.ShapeDtypeStruct((npass * NC * NPAD, D), jnp.float32),
        mesh=_mesh,
        compiler_params=pltpu.CompilerParams(needs_layout_passes=False),
        scratch_types=scratch,
    )
    def prop(*refs):
        ys = refs[:npass]
        src3, dst3, zero_hbm, out_hbm = refs[npass:npass + 4]
        acc, src2, dst2, rows, sem0, sem1, sem2, sem3 = refs[npass + 4:]
        c = lax.axis_index("c")
        s = lax.axis_index("s")
        w = s * NC + c
        pltpu.sync_copy(src3.at[w], src2)

        for k in range(npass):
            y = ys[k]
            pltpu.sync_copy(zero_hbm, acc.at[pl.ds(s * RPT, RPT)])
            plsc.subcore_barrier()

            def start_g(g, b, sem):
                pltpu.async_copy(y.at[src2.at[g]], rows.at[b], sem)

            def wait_g(b, sem):
                pltpu.make_async_copy(y.at[src2.at[0]], rows.at[b],
                                      sem).wait()

            def start_s(j, b, sem):
                pltpu.async_copy(rows.at[b], acc.at[dst2.at[j]], sem,
                                 add=True)

            def wait_s(b, sem):
                pltpu.make_async_copy(rows.at[b], acc.at[dst2.at[0]],
                                      sem).wait()

            def step(g0, j0):
                wait_g(0, sem0)
                start_s(j0, 0, sem2)
                wait_g(1, sem1)
                start_s(j0 + 1, 1, sem3)
                wait_s(0, sem2)
                start_g(g0 + 2, 0, sem0)
                wait_s(1, sem3)
                start_g(g0 + 3, 1, sem1)

            def pair_drain(j0):
                wait_g(0, sem0)
                start_s(j0, 0, sem2)
                wait_g(1, sem1)
                start_s(j0 + 1, 1, sem3)
                wait_s(0, sem2)
                wait_s(1, sem3)

            half = NCHUNK // 2
            qp = half // 2 - 1
            pltpu.sync_copy(dst3.at[w, pl.ds(0, half)], dst2)
            start_g(0, 0, sem0)
            start_g(1, 1, sem1)

            def body_a(p, carry):
                step(2 * p, 2 * p)
                return carry

            lax.fori_loop(0, qp, body_a, 0)
            pair_drain(half - 2)
            pltpu.sync_copy(dst3.at[w, pl.ds(half, half)], dst2)
            start_g(half, 0, sem0)
            start_g(half + 1, 1, sem1)

            def body_b(p, carry):
                step(half + 2 * p, 2 * p)
                return carry

            lax.fori_loop(0, qp, body_b, 0)
            pair_drain(half - 2)
            plsc.subcore_barrier()
            pltpu.sync_copy(
                acc.at[pl.ds(s * RPT, RPT)],
                out_hbm.at[pl.ds((k * NC + c) * NPAD + s * RPT, RPT)])

    return prop


_prop1 = _make_prop_es(1)


EPW1 = EPAD // NS
NCHUNK1 = EPW1 // CHUNK


def _make_prop_ps(npass):
    scratch = [
        pltpu.VMEM_SHARED((NPAD, D), jnp.float32),
        pltpu.VMEM((NCHUNK1 // 2, CHUNK), jnp.int32),
        pltpu.VMEM((NCHUNK1 // 4, CHUNK), jnp.int32),
        pltpu.VMEM((2, CHUNK, D), jnp.float32),
        pltpu.SemaphoreType.DMA,
        pltpu.SemaphoreType.DMA,
    ]

    @functools.partial(
        pl.kernel,
        out_type=jax.ShapeDtypeStruct((npass * NPAD, D), jnp.float32),
        mesh=_mesh,
        compiler_params=pltpu.CompilerParams(needs_layout_passes=False),
        scratch_types=scratch,
    )
    def prop(*refs):
        ys = refs[:npass]
        src3, dst3, zero_hbm, out_hbm = refs[npass:npass + 4]
        acc, src2, dst2, rows, sem0, sem1 = refs[npass + 4:]
        c = lax.axis_index("c")
        s = lax.axis_index("s")

        def run_passes(klist):
            for k in klist:
                y = ys[k]
                pltpu.sync_copy(zero_hbm, acc.at[pl.ds(s * RPT, RPT)])
                plsc.subcore_barrier()

                def start_g(gl, b, sem):
                    pltpu.async_copy(y.at[src2.at[gl]], rows.at[b], sem)

                def wait_g(b, sem):
                    pltpu.make_async_copy(y.at[src2.at[0]], rows.at[b],
                                          sem).wait()

                def scat(j, b):
                    pltpu.sync_copy(rows.at[b], acc.at[dst2.at[j]],
                                    add=True)

                def step(gl0, j0):
                    wait_g(0, sem0)
                    scat(j0, 0)
                    start_g(gl0 + 2, 0, sem0)
                    wait_g(1, sem1)
                    scat(j0 + 1, 1)
                    start_g(gl0 + 3, 1, sem1)

                for q in range(4):
                    if q % 2 == 0:
                        pltpu.sync_copy(
                            src3.at[s, pl.ds((q // 2) * 80, 80)], src2)
                    pltpu.sync_copy(dst3.at[s, pl.ds(q * 40, 40)], dst2)
                    loc = (q % 2) * 40
                    start_g(loc, 0, sem0)
                    start_g(loc + 1, 1, sem1)

                    def body(p, carry, loc=loc):
                        step(loc + 2 * p, 2 * p)
                        return carry

                    lax.fori_loop(0, 19, body, 0)
                    wait_g(0, sem0)
                    scat(38, 0)
                    wait_g(1, sem1)
                    scat(39, 1)
                plsc.subcore_barrier()
                pltpu.sync_copy(
                    acc.at[pl.ds(s * RPT, RPT)],
                    out_hbm.at[pl.ds(k * NPAD + s * RPT, RPT)])

        @pl.when(c == 0)
        def _():
            run_passes(range(npass // 2))

        @pl.when(c == 1)
        def _():
            run_passes(range(npass // 2, npass))

    return prop


_prop2 = _make_prop_ps(2)
_prop4 = _make_prop_ps(4)


RPW = NPAD // NW


@functools.partial(
    pl.kernel,
    out_type=jax.ShapeDtypeStruct((NW, G + 1, 4 * D), jnp.float32),
    mesh=_mesh,
    compiler_params=pltpu.CompilerParams(needs_layout_passes=False),
    scratch_types=[
        pltpu.VMEM((RPW,), jnp.int32),
        pltpu.VMEM((POOL_RB, 4 * D), jnp.float32),
        pltpu.VMEM((G + 1, 4 * D), jnp.float32),
    ],
)
def _sc_pool(h3_hbm, batch_hbm, out_hbm, bchunk, rbuf, acc):
    c = lax.axis_index("c")
    s = lax.axis_index("s")
    w = s * NC + c
    pltpu.sync_copy(batch_hbm.at[pl.ds(w * RPW, RPW)], bchunk)
    iota = lax.iota(jnp.int32, 16)
    zeros16 = jnp.zeros((16,), jnp.float32)
    for g in range(G + 1):
        for j in range(4 * D // 16):
            acc[g, pl.ds(j * 16, 16)] = zeros16

    def chunk_body(ci, carry):
        r0 = ci * POOL_RB
        pltpu.sync_copy(h3_hbm.at[pl.ds(w * RPW + r0, POOL_RB)], rbuf)

        def row_body(i, carry2):
            li = r0 + i
            seg = plsc.load_gather(bchunk, [jnp.full((16,), li, jnp.int32)])
            for j in range(4 * D // 16):
                col = j * 16 + iota
                rv = rbuf[i, pl.ds(j * 16, 16)]
                cur = plsc.load_gather(acc, [seg, col])
                plsc.store_scatter(acc, [seg, col], jnp.maximum(cur, rv))
            return carry2

        lax.fori_loop(0, POOL_RB, row_body, 0)
        return carry

    lax.fori_loop(0, RPW // POOL_RB, chunk_body, 0)
    pltpu.sync_copy(acc, out_hbm.at[w])


def _tc_first(x_pad, cnt_t, W1):
    def body(x_ref, cnt_ref, w_ref, y1_ref, dis_ref):
        i = pl.program_id(0)
        rows = lax.broadcasted_iota(jnp.int32, (BM, 1), 0) + i * BM
        deg = cnt_ref[:, 0:1] + cnt_ref[:, 1:2] + 1.0
        dis = lax.rsqrt(deg) * jnp.where(rows < N, 1.0, 0.0)
        dis_ref[...] = dis
        h = jnp.dot(x_ref[...], w_ref[...],
                    preferred_element_type=jnp.float32)
        y1_ref[...] = h * dis

    return pl.pallas_call(
        body,
        grid=(NPAD // BM,),
        in_specs=[
            pl.BlockSpec((BM, D), lambda i: (i, 0)),
            pl.BlockSpec((BM, 2), lambda i: (i, 0)),
            pl.BlockSpec((D, D), lambda i: (0, 0)),
        ],
        out_specs=[
            pl.BlockSpec((BM, D), lambda i: (i, 0)),
            pl.BlockSpec((BM, 1), lambda i: (i, 0)),
        ],
        out_shape=[
            jax.ShapeDtypeStruct((NPAD, D), jnp.float32),
            jax.ShapeDtypeStruct((NPAD, 1), jnp.float32),
        ],
    )(x_pad, cnt_t, W1)


def _tc_mid1(Sa, Sb, y1, dis, b1, W2):
    def body(sa_ref, sb_ref, y_ref, dis_ref, b_ref, w_ref, out_ref):
        d = dis_ref[...]
        h1 = jnp.maximum(
            (sa_ref[...] + sb_ref[...] + y_ref[...]) * d + b_ref[...], 0.0)
        y2 = jnp.dot(h1, w_ref[...], preferred_element_type=jnp.float32) * d
        out_ref[0] = y2[:, 0:D]
        out_ref[1] = y2[:, D:2 * D]

    return pl.pallas_call(
        body,
        grid=(NPAD // BM,),
        in_specs=[
            pl.BlockSpec((BM, D), lambda i: (i, 0)),
            pl.BlockSpec((BM, D), lambda i: (i, 0)),
            pl.BlockSpec((BM, D), lambda i: (i, 0)),
            pl.BlockSpec((BM, 1), lambda i: (i, 0)),
            pl.BlockSpec((1, D), lambda i: (0, 0)),
            pl.BlockSpec((D, 2 * D), lambda i: (0, 0)),
        ],
        out_specs=pl.BlockSpec((2, BM, D), lambda i: (0, i, 0)),
        out_shape=jax.ShapeDtypeStruct((2, NPAD, D), jnp.float32),
    )(Sa, Sb, y1, dis, b1, W2)


def _tc_mid2(S2, y2, dis, b2, W3):
    def body(s_ref, y_ref, dis_ref, b_ref, w_ref, out_ref):
        d = dis_ref[...]
        z = jnp.concatenate(
            [(s_ref[0] + y_ref[0]) * d,
             (s_ref[1] + y_ref[1]) * d], axis=1)
        h2 = jnp.maximum(z + b_ref[...], 0.0)
        y3 = jnp.dot(h2, w_ref[...], preferred_element_type=jnp.float32) * d
        for k in range(4):
            out_ref[k] = y3[:, k * D:(k + 1) * D]

    return pl.pallas_call(
        body,
        grid=(NPAD // BM,),
        in_specs=[
            pl.BlockSpec((2, BM, D), lambda i: (0, i, 0)),
            pl.BlockSpec((2, BM, D), lambda i: (0, i, 0)),
            pl.BlockSpec((BM, 1), lambda i: (i, 0)),
            pl.BlockSpec((1, 2 * D), lambda i: (0, 0)),
            pl.BlockSpec((2 * D, 4 * D), lambda i: (0, 0)),
        ],
        out_specs=pl.BlockSpec((4, BM, D), lambda i: (0, i, 0)),
        out_shape=jax.ShapeDtypeStruct((4, NPAD, D), jnp.float32),
    )(S2, y2, dis, b2, W3)


def _tc_last(S3, y3, dis, b3):
    def body(s_ref, y_ref, dis_ref, b_ref, out_ref):
        d = dis_ref[...]
        z = jnp.concatenate(
            [(s_ref[k] + y_ref[k]) * d for k in range(4)],
            axis=1)
        out_ref[...] = jnp.maximum(z + b_ref[...], 0.0)

    return pl.pallas_call(
        body,
        grid=(NPAD // BM,),
        in_specs=[
            pl.BlockSpec((4, BM, D), lambda i: (0, i, 0)),
            pl.BlockSpec((4, BM, D), lambda i: (0, i, 0)),
            pl.BlockSpec((BM, 1), lambda i: (i, 0)),
            pl.BlockSpec((1, 4 * D), lambda i: (0, 0)),
        ],
        out_specs=pl.BlockSpec((BM, 4 * D), lambda i: (i, 0)),
        out_shape=jax.ShapeDtypeStruct((NPAD, 4 * D), jnp.float32),
    )(S3, y3, dis, b3)


def _tc_head(parts, Wg1, bg1, Wg2, bg2):
    def body(p_ref, w1_ref, b1_ref, w2_ref, b2_ref, out_ref):
        pooled = jnp.max(p_ref[:, 0:G, :], axis=0)
        g = jnp.maximum(
            jnp.dot(pooled, w1_ref[...],
                    preferred_element_type=jnp.float32) + b1_ref[...], 0.0)
        out_ref[...] = (
            jnp.dot(g, w2_ref[...], preferred_element_type=jnp.float32)
            + b2_ref[...])

    return pl.pallas_call(
        body,
        out_shape=jax.ShapeDtypeStruct((G, 1), jnp.float32),
    )(parts, Wg1, bg1, Wg2, bg2)


def kernel(x, edge_index, batch, W1, b1, W2, b2, W3, b3, Wg1, bg1, Wg2, bg2):
    src = edge_index[0].astype(jnp.int32)
    dst = edge_index[1].astype(jnp.int32)
    epad = jnp.full((EPAD - E,), N, dtype=jnp.int32)
    src_p = jnp.concatenate([src, epad])
    dst_p = jnp.concatenate([dst, epad])
    x_pad = jnp.pad(x, ((0, NPAD - N), (0, 0)))
    zeros1 = jnp.zeros((RPT,), jnp.float32)
    zeros2 = jnp.zeros((RPT, D), jnp.float32)

    batch_p = jnp.concatenate(
        [batch.astype(jnp.int32),
         jnp.full((NPAD - N,), G, dtype=jnp.int32)])

    cnt = _sc_degree(dst_p, zeros1)
    cnt_t = cnt.reshape(NC, NPAD).T

    src3w = src_p.reshape(NW, NCHUNK, CHUNK)
    dst3w = dst_p.reshape(NW, NCHUNK, CHUNK)
    src3s = src_p.reshape(NS, NCHUNK1, CHUNK)
    dst3s = dst_p.reshape(NS, NCHUNK1, CHUNK)

    y1, dis = _tc_first(x_pad, cnt_t, W1)
    S1 = _prop1(y1, src3w, dst3w, zeros2).reshape(NC, NPAD, D)
    y2 = _tc_mid1(S1[0], S1[1], y1, dis, b1.reshape(1, D), W2)

    S2 = _prop2(y2[0], y2[1], src3s, dst3s, zeros2).reshape(2, NPAD, D)
    y3 = _tc_mid2(S2, y2, dis, b2.reshape(1, 2 * D), W3)

    S3 = _prop4(y3[0], y3[1], y3[2], y3[3], src3s, dst3s,
                zeros2).reshape(4, NPAD, D)
    h3 = _tc_last(S3, y3, dis, b3.reshape(1, 4 * D))

    parts = _sc_pool(h3, batch_p)
    return _tc_head(parts, Wg1, bg1.reshape(1, 1024), Wg2,
                    bg2.reshape(1, 1))
